# baseline (speedup 1.0000x reference)
import functools

import jax
import jax.numpy as jnp
from jax.experimental import pallas as pl
from jax.experimental.pallas import tpu as pltpu

_LOG2E = 1.4426950408889634


def _prep_kernel(x_ref, ln_g_ref, ln_b_ref, fc_wt_ref, fc_b_ref, gc_w_ref,
                 xn_ref, qs_ref, sup_ref):
    x = x_ref[...]
    mu = jnp.mean(x, axis=1, keepdims=True)
    var = jnp.mean((x - mu) ** 2, axis=1, keepdims=True)
    x1 = (x - mu) * jax.lax.rsqrt(var + 1e-5) * ln_g_ref[...] + ln_b_ref[...]
    xf = jnp.dot(x1, fc_wt_ref[...], preferred_element_type=jnp.float32)
    xf = xf + fc_b_ref[...]
    norm = jnp.sqrt(jnp.sum(xf * xf, axis=1, keepdims=True))
    xn = xf / jnp.maximum(norm, 1e-12)
    xn_ref[...] = xn.astype(jnp.float8_e4m3fn)
    qs_ref[...] = (xn * _LOG2E).astype(jnp.float8_e4m3fn)
    sup_ref[...] = jnp.dot(x1, gc_w_ref[...],
                           preferred_element_type=jnp.float32).astype(jnp.float8_e4m3fn)


def _attn_kernel(q_ref, k_ref, v_ref, x_ref, gc_b_ref, o_ref):
    q = q_ref[...]
    k = k_ref[...]
    s = jax.lax.dot_general(q, k, (((1,), (1,)), ((), ())),
                            preferred_element_type=jnp.float32)
    e = jnp.exp2(s)
    l = jnp.sum(e, axis=1, keepdims=True)
    p = e.astype(jnp.float8_e4m3fn)
    o = jnp.dot(p, v_ref[...], preferred_element_type=jnp.float32)
    o_ref[...] = o / l + gc_b_ref[...] + x_ref[...]


@functools.partial(jax.jit, static_argnames=("bm_prep", "bm"))
def _run(X, ln_g, ln_b, fc_w, fc_b, gc_w, gc_b, bm_prep=2000, bm=1000):
    N, D = X.shape
    F = fc_w.shape[0]

    ln_g2 = ln_g.reshape(1, D)
    ln_b2 = ln_b.reshape(1, D)
    fc_wt = fc_w.T
    fc_b2 = fc_b.reshape(1, F)
    gc_b2 = gc_b.reshape(1, D)

    x_norm, q_scaled, support = pl.pallas_call(
        _prep_kernel,
        grid=(N // bm_prep,),
        in_specs=[
            pl.BlockSpec((bm_prep, D), lambda i: (i, 0)),
            pl.BlockSpec((1, D), lambda i: (0, 0)),
            pl.BlockSpec((1, D), lambda i: (0, 0)),
            pl.BlockSpec((D, F), lambda i: (0, 0)),
            pl.BlockSpec((1, F), lambda i: (0, 0)),
            pl.BlockSpec((D, D), lambda i: (0, 0)),
        ],
        out_specs=[
            pl.BlockSpec((bm_prep, F), lambda i: (i, 0)),
            pl.BlockSpec((bm_prep, F), lambda i: (i, 0)),
            pl.BlockSpec((bm_prep, D), lambda i: (i, 0)),
        ],
        out_shape=[
            jax.ShapeDtypeStruct((N, F), jnp.float8_e4m3fn),
            jax.ShapeDtypeStruct((N, F), jnp.float8_e4m3fn),
            jax.ShapeDtypeStruct((N, D), jnp.float8_e4m3fn),
        ],
        compiler_params=pltpu.CompilerParams(
            dimension_semantics=("parallel",),
        ),
    )(X, ln_g2, ln_b2, fc_wt, fc_b2, gc_w)

    out = pl.pallas_call(
        _attn_kernel,
        grid=(N // bm,),
        in_specs=[
            pl.BlockSpec((bm, F), lambda i: (i, 0)),
            pl.BlockSpec((N, F), lambda i: (0, 0)),
            pl.BlockSpec((N, D), lambda i: (0, 0)),
            pl.BlockSpec((bm, D), lambda i: (i, 0)),
            pl.BlockSpec((1, D), lambda i: (0, 0)),
        ],
        out_specs=pl.BlockSpec((bm, D), lambda i: (i, 0)),
        out_shape=jax.ShapeDtypeStruct((N, D), jnp.float32),
        compiler_params=pltpu.CompilerParams(
            dimension_semantics=("parallel",),
        ),
    )(q_scaled, x_norm, support, X, gc_b2)

    return out


def kernel(X, ln_g, ln_b, fc_w, fc_b, gc_w, gc_b):
    return _run(X, ln_g, ln_b, fc_w, fc_b, gc_w, gc_b)


# speedup vs baseline: 1.0283x; 1.0283x over previous
import jax
import jax.numpy as jnp
from jax.experimental import pallas as pl
from jax.experimental.pallas import tpu as pltpu

_SQRT_LOG2E = 1.2011224087864498
_BM = 1000


def _gcn_kernel(x_ref, ln_g_ref, ln_b_ref, fc_wt_ref, fc_b_ref, gc_w_ref,
                gcb_ref, out_ref, ks_ref, sup_ref):
    t = pl.program_id(0)

    @pl.when(t == 0)
    def _prep():
        x = x_ref[...]
        mu = jnp.mean(x, axis=1, keepdims=True)
        var = jnp.mean((x - mu) ** 2, axis=1, keepdims=True)
        x1 = (x - mu) * jax.lax.rsqrt(var + 1e-5) * ln_g_ref[...] + ln_b_ref[...]
        x1 = x1.astype(jnp.bfloat16)
        xf = jnp.dot(x1, fc_wt_ref[...], preferred_element_type=jnp.float32)
        xf = xf + fc_b_ref[...]
        norm = jnp.sqrt(jnp.sum(xf * xf, axis=1, keepdims=True))
        xn = xf * (_SQRT_LOG2E / jnp.maximum(norm, 1e-12))
        ks_ref[...] = xn.astype(jnp.float8_e4m3fn)
        sup_ref[...] = jnp.dot(
            x1, gc_w_ref[...],
            preferred_element_type=jnp.float32).astype(jnp.float8_e4m3fn)

    @pl.when(t > 0)
    def _attend():
        i = t - 1
        q = ks_ref[pl.ds(i * _BM, _BM), :]
        s = jax.lax.dot_general(q, ks_ref[...], (((1,), (1,)), ((), ())),
                                preferred_element_type=jnp.float32)
        e = jnp.exp2(s)
        l = jnp.sum(e, axis=1, keepdims=True)
        p = e.astype(jnp.float8_e4m3fn)
        o = jnp.dot(p, sup_ref[...], preferred_element_type=jnp.float32)
        out_ref[...] = (o / l + gcb_ref[...]
                        + x_ref[pl.ds(i * _BM, _BM), :])


@jax.jit
def _run(X, ln_g, ln_b, fc_w, fc_b, gc_w, gc_b):
    N, D = X.shape
    F = fc_w.shape[0]

    ln_g2 = ln_g.reshape(1, D)
    ln_b2 = ln_b.reshape(1, D)
    fc_wt = fc_w.T.astype(jnp.bfloat16)
    fc_b2 = fc_b.reshape(1, F)
    gc_wh = gc_w.astype(jnp.bfloat16)
    gc_b2 = gc_b.reshape(1, D)

    nblocks = N // _BM
    out = pl.pallas_call(
        _gcn_kernel,
        grid=(nblocks + 1,),
        in_specs=[
            pl.BlockSpec((N, D), lambda i: (0, 0)),
            pl.BlockSpec((1, D), lambda i: (0, 0)),
            pl.BlockSpec((1, D), lambda i: (0, 0)),
            pl.BlockSpec((D, F), lambda i: (0, 0)),
            pl.BlockSpec((1, F), lambda i: (0, 0)),
            pl.BlockSpec((D, D), lambda i: (0, 0)),
            pl.BlockSpec((1, D), lambda i: (0, 0)),
        ],
        out_specs=pl.BlockSpec(
            (_BM, D), lambda i: (jnp.maximum(i - 1, 0), 0)),
        out_shape=jax.ShapeDtypeStruct((N, D), jnp.float32),
        scratch_shapes=[
            pltpu.VMEM((N, F), jnp.float8_e4m3fn),
            pltpu.VMEM((N, D), jnp.float8_e4m3fn),
        ],
        compiler_params=pltpu.CompilerParams(
            dimension_semantics=("arbitrary",),
        ),
    )(X, ln_g2, ln_b2, fc_wt, fc_b2, gc_wh, gc_b2)

    return out


def kernel(X, ln_g, ln_b, fc_w, fc_b, gc_w, gc_b):
    return _run(X, ln_g, ln_b, fc_w, fc_b, gc_w, gc_b)
